# baseline (speedup 1.0000x reference)
import functools
import math

import jax
import jax.numpy as jnp
from jax import lax
from jax.experimental import pallas as pl
from jax.experimental.pallas import tpu as pltpu
from jax.experimental.pallas import tpu_sc as plsc

N_EMB = 64
SCALE = math.sqrt(N_EMB)

NC = 2
NS = 16
NW = NC * NS


@functools.partial(jax.jit, static_argnums=(2, 3))
def _embed(xf, lut, B, C):
    D = N_EMB
    b_per_w = B // NW
    n_chunks = b_per_w // C
    mesh = plsc.VectorSubcoreMesh(core_axis_name="c", subcore_axis_name="s")

    @functools.partial(
        pl.kernel,
        out_type=jax.ShapeDtypeStruct((B, D), jnp.float32),
        mesh=mesh,
        scratch_types=[
            pltpu.VMEM((C,), jnp.int32),
            pltpu.VMEM((C, D), jnp.float32),
            pltpu.SemaphoreType.DMA,
        ],
        compiler_params=pltpu.CompilerParams(use_tc_tiling_on_sc=False),
    )
    def k(x_hbm, lut_hbm, out_hbm, idx_v, rows_v, sem):
        wid = lax.axis_index("s") * NC + lax.axis_index("c")
        base = wid * b_per_w

        def chunk_body(i, carry):
            off = base + i * C
            pltpu.sync_copy(x_hbm.at[pl.ds(off, C)], idx_v)
            pltpu.async_copy(lut_hbm.at[idx_v], rows_v, sem).wait()

            def scale_row(r, carry2):
                for j in range(D // 16):
                    sl = pl.ds(j * 16, 16)
                    rows_v[r, sl] = rows_v[r, sl] * SCALE
                return carry2

            lax.fori_loop(0, C, scale_row, 0)
            pltpu.sync_copy(rows_v, out_hbm.at[pl.ds(off, C)])
            return carry

        lax.fori_loop(0, n_chunks, chunk_body, 0)

    return k(xf, lut)


def kernel(x, lut):
    B = x.shape[0] * x.shape[1]
    xf = x.reshape(B).astype(jnp.int32)
    out = _embed(xf, lut, B, 512)
    return out.reshape(x.shape[0], x.shape[1], N_EMB)


# speedup vs baseline: 1.1371x; 1.1371x over previous
import functools
import math

import jax
import jax.numpy as jnp
from jax import lax
from jax.experimental import pallas as pl
from jax.experimental.pallas import tpu as pltpu
from jax.experimental.pallas import tpu_sc as plsc

N_EMB = 64
SCALE = math.sqrt(N_EMB)

NC = 2
NS = 16
NW = NC * NS

NBUF = 4
LOOKAHEAD = 2


@functools.partial(jax.jit, static_argnums=(2, 3))
def _embed(xf, lut, B, C):
    D = N_EMB
    b_per_w = B // NW
    n_chunks = b_per_w // C
    assert n_chunks % NBUF == 0
    mesh = plsc.VectorSubcoreMesh(core_axis_name="c", subcore_axis_name="s")

    @functools.partial(
        pl.kernel,
        out_type=jax.ShapeDtypeStruct((B, D), jnp.float32),
        mesh=mesh,
        scratch_types=[
            pltpu.VMEM((b_per_w,), jnp.int32),
            [pltpu.VMEM((C, D), jnp.float32) for _ in range(NBUF)],
            [pltpu.SemaphoreType.DMA for _ in range(NBUF)],
            [pltpu.SemaphoreType.DMA for _ in range(NBUF)],
        ],
        compiler_params=pltpu.CompilerParams(use_tc_tiling_on_sc=False),
    )
    def k(x_hbm, lut_hbm, out_hbm, idx_v, rows, gsem, ssem):
        wid = lax.axis_index("s") * NC + lax.axis_index("c")
        base = wid * b_per_w

        pltpu.sync_copy(x_hbm.at[pl.ds(base, b_per_w)], idx_v)

        def start_gather(c, b):
            pltpu.async_copy(
                lut_hbm.at[idx_v.at[pl.ds(c * C, C)]], rows[b], gsem[b])

        def wait_gather(c, b):
            pltpu.make_async_copy(
                lut_hbm.at[idx_v.at[pl.ds(c * C, C)]], rows[b], gsem[b]).wait()

        def start_scatter(c, b):
            pltpu.async_copy(
                rows[b], out_hbm.at[pl.ds(base + c * C, C)], ssem[b])

        def wait_scatter(c, b):
            pltpu.make_async_copy(
                rows[b], out_hbm.at[pl.ds(base + c * C, C)], ssem[b]).wait()

        for j in range(LOOKAHEAD):
            start_gather(j, j)

        def outer(io, carry):
            for b in range(NBUF):
                i = io * NBUF + b
                nb = (b + LOOKAHEAD) % NBUF

                @pl.when(i + LOOKAHEAD < n_chunks)
                def _():
                    @pl.when(i + LOOKAHEAD >= NBUF)
                    def _():
                        wait_scatter(i + LOOKAHEAD - NBUF, nb)
                    start_gather(i + LOOKAHEAD, nb)

                wait_gather(i, b)

                @plsc.parallel_loop(0, C, step=1, unroll=8)
                def _(r):
                    for j in range(D // 16):
                        sl = pl.ds(j * 16, 16)
                        rows[b][r, sl] = rows[b][r, sl] * SCALE

                start_scatter(i, b)
            return carry

        lax.fori_loop(0, n_chunks // NBUF, outer, 0)

        for b in range(NBUF):
            wait_scatter(n_chunks - NBUF + b, b)

    return k(xf, lut)


def kernel(x, lut):
    B = x.shape[0] * x.shape[1]
    xf = x.reshape(B).astype(jnp.int32)
    out = _embed(xf, lut, B, 400)
    return out.reshape(x.shape[0], x.shape[1], N_EMB)
